# baseline (speedup 1.0000x reference)
import functools

import jax
import jax.numpy as jnp
import numpy as np
from jax import lax
from jax.experimental import pallas as pl
from jax.experimental.pallas import tpu as pltpu
from jax.experimental.pallas import tpu_sc as plsc

_FEATURE_DIMS = [38462] * 26
_V = int(sum(_FEATURE_DIMS))
_B = 16384
_NF = len(_FEATURE_DIMS)
_F = _NF + 1
_NC, _NS, _L = 2, 16, 16
_NW = _NC * _NS
_BPW = _B // _NW
_OFFSETS = np.concatenate([[0], np.cumsum(_FEATURE_DIMS)[:-1]]).astype(np.int32)


def _sc_body(idx_hbm, tab_hbm, out_hbm, idx_v, vals_v, out_v, sem):
    wid = lax.axis_index("s") * _NC + lax.axis_index("c")
    base = wid * (_F * _BPW)
    pltpu.sync_copy(idx_hbm.at[pl.ds(base, _F * _BPW)], idx_v)
    pltpu.async_copy(tab_hbm.at[idx_v], vals_v, sem).wait()

    def chunk(c, carry):
        acc = vals_v[pl.ds(c * _L, _L)]
        for f in range(1, _F):
            acc = acc + vals_v[pl.ds(f * _BPW + c * _L, _L)]
        out_v[pl.ds(c * _L, _L)] = acc
        return carry

    lax.fori_loop(0, _BPW // _L, chunk, 0)
    pltpu.sync_copy(out_v, out_hbm.at[pl.ds(wid * _BPW, _BPW)])


_sc_call = pl.kernel(
    _sc_body,
    out_type=jax.ShapeDtypeStruct((_B,), jnp.float32),
    mesh=plsc.VectorSubcoreMesh(
        core_axis_name="c", subcore_axis_name="s",
        num_cores=_NC, num_subcores=_NS,
    ),
    scratch_types=[
        pltpu.VMEM((_F * _BPW,), jnp.int32),
        pltpu.VMEM((_F * _BPW,), jnp.float32),
        pltpu.VMEM((_BPW,), jnp.float32),
        pltpu.SemaphoreType.DMA,
    ],
)


def kernel(data, table, bias):
    offset = jnp.asarray(_OFFSETS, jnp.int32)
    idx = data.astype(jnp.int32) + offset[None, :]
    idx_t = idx.T.reshape(_NF, _NW, _BPW)
    bias_idx = jnp.full((1, _NW, _BPW), _V, jnp.int32)
    idx_aug = jnp.concatenate([idx_t, bias_idx], axis=0)
    idx_flat = idx_aug.transpose(1, 0, 2).reshape(_NW * _F * _BPW)
    tab_aug = jnp.concatenate([table.reshape(-1), bias.reshape(-1)])
    out = _sc_call(idx_flat, tab_aug)
    return out.reshape(_B, 1)


# speedup vs baseline: 2.0884x; 2.0884x over previous
import jax
import jax.numpy as jnp
from jax import lax
from jax.experimental import pallas as pl
from jax.experimental.pallas import tpu as pltpu
from jax.experimental.pallas import tpu_sc as plsc

_VOCAB = 38462
_NF = 26
_B = 16384
_NC, _NS, _L = 2, 16, 16
_BPH = _B // _NC
_BPT = _BPH // _NS


def _sc_body(idx_hbm, tab_hbm, out_hbm,
             subt0, subt1, idx0, idx1, part_v, red_v, res_v,
             shared, sem0, sem1, sem2, sem3):
    s = lax.axis_index("s")
    h = lax.axis_index("c")
    base = h * _BPH

    f0 = s
    f1 = s + _NS
    has2 = f1 < _NF

    cp_t0 = pltpu.async_copy(tab_hbm.at[f0], subt0, sem0)
    cp_i0 = pltpu.async_copy(idx_hbm.at[f0, pl.ds(base, _BPH)], idx0, sem1)

    @pl.when(has2)
    def _():
        pltpu.async_copy(tab_hbm.at[f1], subt1, sem2).wait()
        pltpu.async_copy(idx_hbm.at[f1, pl.ds(base, _BPH)], idx1, sem3).wait()

    cp_t0.wait()
    cp_i0.wait()

    def acc0(c, carry):
        ids = idx0[pl.ds(c * _L, _L)]
        part_v[pl.ds(c * _L, _L)] = plsc.load_gather(subt0, [ids])
        return carry

    lax.fori_loop(0, _BPH // _L, acc0, 0)

    @pl.when(has2)
    def _():
        def acc1(c, carry):
            ids = idx1[pl.ds(c * _L, _L)]
            part_v[pl.ds(c * _L, _L)] = (
                part_v[pl.ds(c * _L, _L)] + plsc.load_gather(subt1, [ids]))
            return carry

        lax.fori_loop(0, _BPH // _L, acc1, 0)

    pltpu.sync_copy(part_v, shared.at[s])
    plsc.subcore_barrier()
    pltpu.sync_copy(shared.at[:, pl.ds(s * _BPT, _BPT)], red_v)

    def red(c, carry):
        acc = red_v[0, pl.ds(c * _L, _L)]
        for t in range(1, _NS):
            acc = acc + red_v[t, pl.ds(c * _L, _L)]
        res_v[pl.ds(c * _L, _L)] = acc
        return carry

    lax.fori_loop(0, _BPT // _L, red, 0)
    pltpu.sync_copy(res_v, out_hbm.at[pl.ds(base + s * _BPT, _BPT)])


_sc_call = pl.kernel(
    _sc_body,
    out_type=jax.ShapeDtypeStruct((_B,), jnp.float32),
    mesh=plsc.VectorSubcoreMesh(
        core_axis_name="c", subcore_axis_name="s",
        num_cores=_NC, num_subcores=_NS,
    ),
    scratch_types=[
        pltpu.VMEM((_VOCAB,), jnp.float32),
        pltpu.VMEM((_VOCAB,), jnp.float32),
        pltpu.VMEM((_BPH,), jnp.int32),
        pltpu.VMEM((_BPH,), jnp.int32),
        pltpu.VMEM((_BPH,), jnp.float32),
        pltpu.VMEM((_NS, _BPT), jnp.float32),
        pltpu.VMEM((_BPT,), jnp.float32),
        pltpu.VMEM_SHARED((_NS, _BPH), jnp.float32),
        pltpu.SemaphoreType.DMA,
        pltpu.SemaphoreType.DMA,
        pltpu.SemaphoreType.DMA,
        pltpu.SemaphoreType.DMA,
    ],
    compiler_params=pltpu.CompilerParams(needs_layout_passes=False),
)


def kernel(data, table, bias):
    idx_t = data.astype(jnp.int32).T
    tab2 = table.reshape(_NF, _VOCAB)
    out = _sc_call(idx_t, tab2)
    return out.reshape(_B, 1) + bias
